# baseline (speedup 1.0000x reference)
import functools

import jax
import jax.numpy as jnp
from jax import lax
from jax.experimental import pallas as pl
from jax.experimental.pallas import tpu as pltpu
from jax.experimental.pallas import tpu_sc as plsc

N = 10000
E = 320000
D_IN = 128
H1 = 64
H2 = 32
C = 40
CP = 48

NC = 2
NS = 16
NW = NC * NS
CH = 64
CPT = 160
EP = NW * CPT * CH
NBUF = 2
WAVES = CPT // NBUF - 1
NP = 10240
RPT = NP // NS
ZR = RPT // 5
KH = 8

ROWS_BLK = 1000
GRID = N // ROWS_BLK


@functools.lru_cache(maxsize=None)
def _make_hist():
    mesh = plsc.VectorSubcoreMesh(core_axis_name="c", subcore_axis_name="s",
                                  num_cores=NC, num_subcores=NS)
    return functools.partial(
        pl.kernel,
        out_type=jax.ShapeDtypeStruct((NC, NP, 16), jnp.float32),
        mesh=mesh,
        compiler_params=pltpu.CompilerParams(use_tc_tiling_on_sc=False),
        scratch_types=[
            pltpu.VMEM((CPT, CH), jnp.int32),
            pltpu.VMEM((RPT, 16), jnp.float32),
            pltpu.VMEM((CH, 16), jnp.float32),
            pltpu.VMEM_SHARED((NP, 16), jnp.float32),
            pltpu.SemaphoreType.DMA,
            pltpu.SemaphoreType.DMA,
        ],
    )(_hist_body)


def _hist_body(dst_hbm, out_hbm, idx_v, zbuf, ones_v, acc, sem, ssem):
    c = lax.axis_index("c")
    s = lax.axis_index("s")
    wid = c * NS + s

    @pl.loop(0, RPT)
    def _(r):
        zbuf[r] = jnp.zeros((16,), jnp.float32)

    @pl.loop(0, CH)
    def _(r):
        ones_v[r] = jnp.ones((16,), jnp.float32)

    pltpu.sync_copy(zbuf, acc.at[pl.ds(s * RPT, RPT)])
    pltpu.async_copy(dst_hbm.at[wid], idx_v, sem).wait()
    plsc.subcore_barrier()

    @pl.loop(0, CPT)
    def _(i):
        pltpu.sync_copy(ones_v, acc.at[idx_v.at[i]], add=True)

    plsc.subcore_barrier()
    pltpu.sync_copy(acc.at[pl.ds(s * RPT, RPT)],
                    out_hbm.at[c].at[pl.ds(s * RPT, RPT)])


@functools.lru_cache(maxsize=None)
def _make_propagate(D):
    mesh = plsc.VectorSubcoreMesh(core_axis_name="c", subcore_axis_name="s",
                                  num_cores=NC, num_subcores=NS)

    @functools.partial(
        pl.kernel,
        out_type=jax.ShapeDtypeStruct((NC, NP, D), jnp.float32),
        mesh=mesh,
        compiler_params=pltpu.CompilerParams(use_tc_tiling_on_sc=False),
        scratch_types=[
            pltpu.VMEM((CPT, CH), jnp.int32),
            pltpu.VMEM((CPT, CH), jnp.int32),
            pltpu.VMEM((CH, D), jnp.float32),
            pltpu.VMEM((CH, D), jnp.float32),
            pltpu.VMEM((ZR, D), jnp.float32),
            pltpu.VMEM_SHARED((NP, D), jnp.float32),
            pltpu.SemaphoreType.DMA,
            pltpu.SemaphoreType.DMA,
        ],
    )
    def _prop(h_hbm, src_hbm, dst_hbm, out_hbm, srci, dsti, b0, b1,
              zbuf, acc, g0, g1):
        bufs = (b0, b1)
        gsems = (g0, g1)
        c = lax.axis_index("c")
        s = lax.axis_index("s")
        wid = c * NS + s

        @pl.loop(0, ZR)
        def _(r):
            @pl.loop(0, D // 16)
            def _(j):
                zbuf[r, pl.ds(j * 16, 16)] = jnp.zeros((16,), jnp.float32)

        pltpu.sync_copy(src_hbm.at[wid], srci)
        pltpu.sync_copy(dst_hbm.at[wid], dsti)

        @pl.loop(0, 5)
        def _(j):
            pltpu.sync_copy(zbuf, acc.at[pl.ds(s * RPT + j * ZR, ZR)])

        plsc.subcore_barrier()

        for x in range(NBUF):
            pltpu.async_copy(h_hbm.at[srci.at[x]], bufs[x], gsems[x])

        @pl.loop(0, WAVES)
        def _(k):
            base = k * NBUF
            for x in range(NBUF):
                pltpu.make_async_copy(
                    h_hbm.at[srci.at[base + x]], bufs[x], gsems[x]).wait()
                pltpu.sync_copy(bufs[x], acc.at[dsti.at[base + x]], add=True)
                pltpu.async_copy(
                    h_hbm.at[srci.at[base + NBUF + x]], bufs[x], gsems[x])

        fbase = WAVES * NBUF
        for x in range(NBUF):
            pltpu.make_async_copy(
                h_hbm.at[srci.at[fbase + x]], bufs[x], gsems[x]).wait()
            pltpu.sync_copy(bufs[x], acc.at[dsti.at[fbase + x]], add=True)

        plsc.subcore_barrier()
        pltpu.sync_copy(acc.at[pl.ds(s * RPT, RPT)],
                        out_hbm.at[c].at[pl.ds(s * RPT, RPT)])

    return _prop


def _dis_block(deg_ref):
    deg = deg_ref[0, :, 0] + deg_ref[1, :, 0] + 1.0
    return lax.rsqrt(deg)[:, None]


def _in_body(x_ref, deg_ref, w_ref, o_ref):
    dis = _dis_block(deg_ref)
    o_ref[...] = jnp.dot(x_ref[...] * dis, w_ref[...],
                         preferred_element_type=jnp.float32)


def _in_layer(x, degp, w):
    k = x.shape[1]
    m = w.shape[1]
    return pl.pallas_call(
        _in_body,
        grid=(GRID,),
        in_specs=[
            pl.BlockSpec((ROWS_BLK, k), lambda i: (i, 0)),
            pl.BlockSpec((2, ROWS_BLK, 16), lambda i: (0, i, 0)),
            pl.BlockSpec((k, m), lambda i: (0, 0)),
        ],
        out_specs=pl.BlockSpec((ROWS_BLK, m), lambda i: (i, 0)),
        out_shape=jax.ShapeDtypeStruct((N, m), jnp.float32),
    )(x, degp, w)


def _layer_body(sp_ref, hp_ref, deg_ref, b_ref, w_ref, o_ref):
    dis = _dis_block(deg_ref)
    z = dis * (sp_ref[0] + sp_ref[1] + hp_ref[...]) + b_ref[...]
    z = jnp.maximum(z, 0.0)
    o_ref[...] = jnp.dot(z, w_ref[...], preferred_element_type=jnp.float32) * dis


def _layer(sp, hp, degp, b, w):
    d_in = hp.shape[1]
    d_out = w.shape[1]
    return pl.pallas_call(
        _layer_body,
        grid=(GRID,),
        in_specs=[
            pl.BlockSpec((2, ROWS_BLK, d_in), lambda i: (0, i, 0)),
            pl.BlockSpec((ROWS_BLK, d_in), lambda i: (i, 0)),
            pl.BlockSpec((2, ROWS_BLK, 16), lambda i: (0, i, 0)),
            pl.BlockSpec((1, d_in), lambda i: (0, 0)),
            pl.BlockSpec((d_in, d_out), lambda i: (0, 0)),
        ],
        out_specs=pl.BlockSpec((ROWS_BLK, d_out), lambda i: (i, 0)),
        out_shape=jax.ShapeDtypeStruct((N, d_out), jnp.float32),
    )(sp, hp, degp, b, w)


def _final_body(sp_ref, hp_ref, deg_ref, b_ref, o_ref):
    dis = _dis_block(deg_ref)
    t = dis * (sp_ref[0] + sp_ref[1] + hp_ref[...])
    t = t[:, :C] + b_ref[...]
    t = t - jnp.max(t, axis=1, keepdims=True)
    e = jnp.exp(t)
    o_ref[...] = e / jnp.sum(e, axis=1, keepdims=True)


def _final(sp, hp, degp, b):
    return pl.pallas_call(
        _final_body,
        grid=(GRID,),
        in_specs=[
            pl.BlockSpec((2, ROWS_BLK, CP), lambda i: (0, i, 0)),
            pl.BlockSpec((ROWS_BLK, CP), lambda i: (i, 0)),
            pl.BlockSpec((2, ROWS_BLK, 16), lambda i: (0, i, 0)),
            pl.BlockSpec((1, C), lambda i: (0, 0)),
        ],
        out_specs=pl.BlockSpec((ROWS_BLK, C), lambda i: (i, 0)),
        out_shape=jax.ShapeDtypeStruct((N, C), jnp.float32),
    )(sp, hp, degp, b)


def kernel(x, edge_index, W1, b1, W2, b2, W3, b3):
    pad = EP - E
    srcp = jnp.concatenate(
        [edge_index[0], jnp.zeros((pad,), jnp.int32)]).reshape(NW, CPT, CH)
    pad_dst = N + (jnp.arange(pad, dtype=jnp.int32) % (NP - N))
    dstp = jnp.concatenate([edge_index[1], pad_dst]).reshape(NW, CPT, CH)
    w3p = jnp.pad(W3, ((0, 0), (0, CP - C)))
    b1r = b1.reshape(1, H1)
    b2r = b2.reshape(1, H2)
    b3r = b3.reshape(1, C)

    degp = _make_hist()(dstp)
    h1p = _in_layer(x, degp, W1)
    s1 = _make_propagate(H1)(h1p, srcp, dstp)
    h2p = _layer(s1, h1p, degp, b1r, W2)
    s2 = _make_propagate(H2)(h2p, srcp, dstp)
    h3p = _layer(s2, h2p, degp, b2r, w3p)
    s3 = _make_propagate(CP)(h3p, srcp, dstp)
    return _final(s3, h3p, degp, b3r)


# speedup vs baseline: 1.9357x; 1.9357x over previous
import functools

import jax
import jax.numpy as jnp
from jax import lax
from jax.experimental import pallas as pl
from jax.experimental.pallas import tpu as pltpu
from jax.experimental.pallas import tpu_sc as plsc

N = 10000
E = 320000
D_IN = 128
H1 = 64
H2 = 32
C = 40
CP = 48

NC = 2
NS = 16
NW = NC * NS
CH = 64
CPT = 160
EP = NW * CPT * CH
NBUF = 2
WAVES = CPT // NBUF - 1
NP = 10240
RPT = NP // NS
ZR = RPT // 5
KH = 8

ROWS_BLK = 1000
GRID = N // ROWS_BLK


@functools.lru_cache(maxsize=None)
def _make_hist():
    mesh = plsc.VectorSubcoreMesh(core_axis_name="c", subcore_axis_name="s",
                                  num_cores=NC, num_subcores=NS)
    return functools.partial(
        pl.kernel,
        out_type=jax.ShapeDtypeStruct((NC, NP, 16), jnp.float32),
        mesh=mesh,
        compiler_params=pltpu.CompilerParams(use_tc_tiling_on_sc=False),
        scratch_types=[
            pltpu.VMEM((CPT, CH), jnp.int32),
            pltpu.VMEM((RPT, 16), jnp.float32),
            pltpu.VMEM((CH, 16), jnp.float32),
            pltpu.VMEM_SHARED((NP, 16), jnp.float32),
            pltpu.SemaphoreType.DMA,
            pltpu.SemaphoreType.DMA,
        ],
    )(_hist_body)


def _hist_body(dst_hbm, out_hbm, idx_v, zbuf, ones_v, acc, sem, ssem):
    c = lax.axis_index("c")
    s = lax.axis_index("s")
    wid = c * NS + s

    @pl.loop(0, RPT)
    def _(r):
        zbuf[r] = jnp.zeros((16,), jnp.float32)

    @pl.loop(0, CH)
    def _(r):
        ones_v[r] = jnp.ones((16,), jnp.float32)

    pltpu.sync_copy(zbuf, acc.at[pl.ds(s * RPT, RPT)])
    pltpu.async_copy(dst_hbm.at[wid], idx_v, sem).wait()
    plsc.subcore_barrier()

    @pl.loop(0, CPT)
    def _(i):
        pltpu.sync_copy(ones_v, acc.at[idx_v.at[i]], add=True)

    plsc.subcore_barrier()
    pltpu.sync_copy(acc.at[pl.ds(s * RPT, RPT)],
                    out_hbm.at[c].at[pl.ds(s * RPT, RPT)])


@functools.lru_cache(maxsize=None)
def _make_propagate(D):
    mesh = plsc.VectorSubcoreMesh(core_axis_name="c", subcore_axis_name="s",
                                  num_cores=NC, num_subcores=NS)

    @functools.partial(
        pl.kernel,
        out_type=jax.ShapeDtypeStruct((NC, NP, D), jnp.float32),
        mesh=mesh,
        compiler_params=pltpu.CompilerParams(use_tc_tiling_on_sc=False),
        scratch_types=[
            pltpu.VMEM((CPT, CH), jnp.int32),
            pltpu.VMEM((CPT, CH), jnp.int32),
            pltpu.VMEM((CH, D), jnp.float32),
            pltpu.VMEM((CH, D), jnp.float32),
            pltpu.VMEM((ZR, D), jnp.float32),
            pltpu.VMEM_SHARED((NP, D), jnp.float32),
            pltpu.SemaphoreType.DMA,
            pltpu.SemaphoreType.DMA,
        ],
    )
    def _prop(h_hbm, src_hbm, dst_hbm, out_hbm, srci, dsti, b0, b1,
              zbuf, acc, g0, g1):
        bufs = (b0, b1)
        gsems = (g0, g1)
        c = lax.axis_index("c")
        s = lax.axis_index("s")
        wid = c * NS + s

        @pl.loop(0, ZR)
        def _(r):
            @pl.loop(0, D // 16)
            def _(j):
                zbuf[r, pl.ds(j * 16, 16)] = jnp.zeros((16,), jnp.float32)

        pltpu.sync_copy(src_hbm.at[wid], srci)
        pltpu.sync_copy(dst_hbm.at[wid], dsti)

        @pl.loop(0, 5)
        def _(j):
            pltpu.sync_copy(zbuf, acc.at[pl.ds(s * RPT + j * ZR, ZR)])

        plsc.subcore_barrier()

        for x in range(NBUF):
            pltpu.async_copy(h_hbm.at[srci.at[x]], bufs[x], gsems[x])

        @pl.loop(0, WAVES)
        def _(k):
            base = k * NBUF
            for x in range(NBUF):
                pltpu.make_async_copy(
                    h_hbm.at[srci.at[base + x]], bufs[x], gsems[x]).wait()
                pltpu.sync_copy(bufs[x], acc.at[dsti.at[base + x]], add=True)
                pltpu.async_copy(
                    h_hbm.at[srci.at[base + NBUF + x]], bufs[x], gsems[x])

        fbase = WAVES * NBUF
        for x in range(NBUF):
            pltpu.make_async_copy(
                h_hbm.at[srci.at[fbase + x]], bufs[x], gsems[x]).wait()
            pltpu.sync_copy(bufs[x], acc.at[dsti.at[fbase + x]], add=True)

        plsc.subcore_barrier()
        pltpu.sync_copy(acc.at[pl.ds(s * RPT, RPT)],
                        out_hbm.at[c].at[pl.ds(s * RPT, RPT)])

    return _prop


def _dis_block(deg_ref):
    deg = deg_ref[0, :, 0] + deg_ref[1, :, 0] + 1.0
    return lax.rsqrt(deg)[:, None]


def _in_body(x_ref, deg_ref, w_ref, o_ref):
    dis = _dis_block(deg_ref)
    o_ref[...] = jnp.dot(x_ref[...] * dis, w_ref[...],
                         preferred_element_type=jnp.float32)


def _in_layer(x, degp, w):
    k = x.shape[1]
    m = w.shape[1]
    return pl.pallas_call(
        _in_body,
        grid=(GRID,),
        in_specs=[
            pl.BlockSpec((ROWS_BLK, k), lambda i: (i, 0)),
            pl.BlockSpec((2, ROWS_BLK, 16), lambda i: (0, i, 0)),
            pl.BlockSpec((k, m), lambda i: (0, 0)),
        ],
        out_specs=pl.BlockSpec((ROWS_BLK, m), lambda i: (i, 0)),
        out_shape=jax.ShapeDtypeStruct((N, m), jnp.float32),
    )(x, degp, w)


def _layer_body(sp_ref, hp_ref, deg_ref, b_ref, w_ref, o_ref):
    dis = _dis_block(deg_ref)
    z = dis * (sp_ref[0] + sp_ref[1] + hp_ref[...]) + b_ref[...]
    z = jnp.maximum(z, 0.0)
    o_ref[...] = jnp.dot(z, w_ref[...], preferred_element_type=jnp.float32) * dis


def _layer(sp, hp, degp, b, w):
    d_in = hp.shape[1]
    d_out = w.shape[1]
    return pl.pallas_call(
        _layer_body,
        grid=(GRID,),
        in_specs=[
            pl.BlockSpec((2, ROWS_BLK, d_in), lambda i: (0, i, 0)),
            pl.BlockSpec((ROWS_BLK, d_in), lambda i: (i, 0)),
            pl.BlockSpec((2, ROWS_BLK, 16), lambda i: (0, i, 0)),
            pl.BlockSpec((1, d_in), lambda i: (0, 0)),
            pl.BlockSpec((d_in, d_out), lambda i: (0, 0)),
        ],
        out_specs=pl.BlockSpec((ROWS_BLK, d_out), lambda i: (i, 0)),
        out_shape=jax.ShapeDtypeStruct((N, d_out), jnp.float32),
    )(sp, hp, degp, b, w)


def _final_body(sp_ref, hp_ref, deg_ref, b_ref, o_ref):
    dis = _dis_block(deg_ref)
    t = dis * (sp_ref[0] + sp_ref[1] + hp_ref[...])
    t = t[:, :C] + b_ref[...]
    t = t - jnp.max(t, axis=1, keepdims=True)
    e = jnp.exp(t)
    o_ref[...] = e / jnp.sum(e, axis=1, keepdims=True)


def _final(sp, hp, degp, b):
    return pl.pallas_call(
        _final_body,
        grid=(GRID,),
        in_specs=[
            pl.BlockSpec((2, ROWS_BLK, CP), lambda i: (0, i, 0)),
            pl.BlockSpec((ROWS_BLK, CP), lambda i: (i, 0)),
            pl.BlockSpec((2, ROWS_BLK, 16), lambda i: (0, i, 0)),
            pl.BlockSpec((1, C), lambda i: (0, 0)),
        ],
        out_specs=pl.BlockSpec((ROWS_BLK, C), lambda i: (i, 0)),
        out_shape=jax.ShapeDtypeStruct((N, C), jnp.float32),
    )(sp, hp, degp, b)


def kernel(x, edge_index, W1, b1, W2, b2, W3, b3):
    pad = EP - E
    pad_src = jnp.arange(pad, dtype=jnp.int32) % N
    srcp = jnp.concatenate([edge_index[0], pad_src]).reshape(NW, CPT, CH)
    pad_dst = N + (jnp.arange(pad, dtype=jnp.int32) % (NP - N))
    dstp = jnp.concatenate([edge_index[1], pad_dst]).reshape(NW, CPT, CH)
    w3p = jnp.pad(W3, ((0, 0), (0, CP - C)))
    b1r = b1.reshape(1, H1)
    b2r = b2.reshape(1, H2)
    b3r = b3.reshape(1, C)

    degp = _make_hist()(dstp)
    h1p = _in_layer(x, degp, W1)
    s1 = _make_propagate(H1)(h1p, srcp, dstp)
    h2p = _layer(s1, h1p, degp, b1r, W2)
    s2 = _make_propagate(H2)(h2p, srcp, dstp)
    h3p = _layer(s2, h2p, degp, b2r, w3p)
    s3 = _make_propagate(CP)(h3p, srcp, dstp)
    return _final(s3, h3p, degp, b3r)


# speedup vs baseline: 2.8235x; 1.4587x over previous
import functools

import jax
import jax.numpy as jnp
from jax import lax
from jax.experimental import pallas as pl
from jax.experimental.pallas import tpu as pltpu
from jax.experimental.pallas import tpu_sc as plsc

N = 10000
E = 320000
D_IN = 128
H1 = 64
H2 = 32
C = 40
CP = 48

NC = 2
NS = 16
NW = NC * NS
CH = 128
CPT = 80
EP = NW * CPT * CH
NBUF = 4
WAVES = CPT // NBUF - 1
NP = 10240
RPT = NP // NS
ZR = RPT // 5
KH = 8

ROWS_BLK = 1000
GRID = N // ROWS_BLK


@functools.lru_cache(maxsize=None)
def _make_hist():
    mesh = plsc.VectorSubcoreMesh(core_axis_name="c", subcore_axis_name="s",
                                  num_cores=NC, num_subcores=NS)
    return functools.partial(
        pl.kernel,
        out_type=jax.ShapeDtypeStruct((NC, NP, 16), jnp.float32),
        mesh=mesh,
        compiler_params=pltpu.CompilerParams(use_tc_tiling_on_sc=False),
        scratch_types=[
            pltpu.VMEM((CPT, CH), jnp.int32),
            pltpu.VMEM((RPT, 16), jnp.float32),
            pltpu.VMEM((CH, 16), jnp.float32),
            pltpu.VMEM_SHARED((NP, 16), jnp.float32),
            pltpu.SemaphoreType.DMA,
            pltpu.SemaphoreType.DMA,
        ],
    )(_hist_body)


def _hist_body(dst_hbm, out_hbm, idx_v, zbuf, ones_v, acc, sem, ssem):
    c = lax.axis_index("c")
    s = lax.axis_index("s")
    wid = c * NS + s

    @pl.loop(0, RPT)
    def _(r):
        zbuf[r] = jnp.zeros((16,), jnp.float32)

    @pl.loop(0, CH)
    def _(r):
        ones_v[r] = jnp.ones((16,), jnp.float32)

    pltpu.sync_copy(zbuf, acc.at[pl.ds(s * RPT, RPT)])
    pltpu.async_copy(dst_hbm.at[wid], idx_v, sem).wait()
    plsc.subcore_barrier()

    @pl.loop(0, CPT)
    def _(i):
        pltpu.sync_copy(ones_v, acc.at[idx_v.at[i]], add=True)

    plsc.subcore_barrier()
    pltpu.sync_copy(acc.at[pl.ds(s * RPT, RPT)],
                    out_hbm.at[c].at[pl.ds(s * RPT, RPT)])


@functools.lru_cache(maxsize=None)
def _make_propagate(D):
    mesh = plsc.VectorSubcoreMesh(core_axis_name="c", subcore_axis_name="s",
                                  num_cores=NC, num_subcores=NS)

    @functools.partial(
        pl.kernel,
        out_type=jax.ShapeDtypeStruct((NC, NP, D), jnp.float32),
        mesh=mesh,
        compiler_params=pltpu.CompilerParams(use_tc_tiling_on_sc=False),
        scratch_types=[
            pltpu.VMEM((CPT, CH), jnp.int32),
            pltpu.VMEM((CPT, CH), jnp.int32),
            pltpu.VMEM((CH, D), jnp.float32),
            pltpu.VMEM((CH, D), jnp.float32),
            pltpu.VMEM((CH, D), jnp.float32),
            pltpu.VMEM((CH, D), jnp.float32),
            pltpu.VMEM((ZR, D), jnp.float32),
            pltpu.VMEM_SHARED((NP, D), jnp.float32),
            pltpu.SemaphoreType.DMA,
            pltpu.SemaphoreType.DMA,
            pltpu.SemaphoreType.DMA,
            pltpu.SemaphoreType.DMA,
        ],
    )
    def _prop(h_hbm, src_hbm, dst_hbm, out_hbm, srci, dsti, b0, b1, b2, b3,
              zbuf, acc, g0, g1, g2, g3):
        bufs = (b0, b1, b2, b3)
        gsems = (g0, g1, g2, g3)
        c = lax.axis_index("c")
        s = lax.axis_index("s")
        wid = c * NS + s

        @pl.loop(0, ZR)
        def _(r):
            @pl.loop(0, D // 16)
            def _(j):
                zbuf[r, pl.ds(j * 16, 16)] = jnp.zeros((16,), jnp.float32)

        pltpu.sync_copy(src_hbm.at[wid], srci)
        pltpu.sync_copy(dst_hbm.at[wid], dsti)

        @pl.loop(0, 5)
        def _(j):
            pltpu.sync_copy(zbuf, acc.at[pl.ds(s * RPT + j * ZR, ZR)])

        plsc.subcore_barrier()

        for x in range(NBUF):
            pltpu.async_copy(h_hbm.at[srci.at[x]], bufs[x], gsems[x])

        @pl.loop(0, WAVES)
        def _(k):
            base = k * NBUF
            for x in range(NBUF):
                pltpu.make_async_copy(
                    h_hbm.at[srci.at[base + x]], bufs[x], gsems[x]).wait()
                pltpu.sync_copy(bufs[x], acc.at[dsti.at[base + x]], add=True)
                pltpu.async_copy(
                    h_hbm.at[srci.at[base + NBUF + x]], bufs[x], gsems[x])

        fbase = WAVES * NBUF
        for x in range(NBUF):
            pltpu.make_async_copy(
                h_hbm.at[srci.at[fbase + x]], bufs[x], gsems[x]).wait()
            pltpu.sync_copy(bufs[x], acc.at[dsti.at[fbase + x]], add=True)

        plsc.subcore_barrier()
        pltpu.sync_copy(acc.at[pl.ds(s * RPT, RPT)],
                        out_hbm.at[c].at[pl.ds(s * RPT, RPT)])

    return _prop


def _dis_block(deg_ref):
    deg = deg_ref[0, :, 0] + deg_ref[1, :, 0] + 1.0
    return lax.rsqrt(deg)[:, None]


def _in_body(x_ref, deg_ref, w_ref, o_ref):
    dis = _dis_block(deg_ref)
    o_ref[...] = jnp.dot(x_ref[...] * dis, w_ref[...],
                         preferred_element_type=jnp.float32)


def _in_layer(x, degp, w):
    k = x.shape[1]
    m = w.shape[1]
    return pl.pallas_call(
        _in_body,
        grid=(GRID,),
        in_specs=[
            pl.BlockSpec((ROWS_BLK, k), lambda i: (i, 0)),
            pl.BlockSpec((2, ROWS_BLK, 16), lambda i: (0, i, 0)),
            pl.BlockSpec((k, m), lambda i: (0, 0)),
        ],
        out_specs=pl.BlockSpec((ROWS_BLK, m), lambda i: (i, 0)),
        out_shape=jax.ShapeDtypeStruct((N, m), jnp.float32),
    )(x, degp, w)


def _layer_body(sp_ref, hp_ref, deg_ref, b_ref, w_ref, o_ref):
    dis = _dis_block(deg_ref)
    z = dis * (sp_ref[0] + sp_ref[1] + hp_ref[...]) + b_ref[...]
    z = jnp.maximum(z, 0.0)
    o_ref[...] = jnp.dot(z, w_ref[...], preferred_element_type=jnp.float32) * dis


def _layer(sp, hp, degp, b, w):
    d_in = hp.shape[1]
    d_out = w.shape[1]
    return pl.pallas_call(
        _layer_body,
        grid=(GRID,),
        in_specs=[
            pl.BlockSpec((2, ROWS_BLK, d_in), lambda i: (0, i, 0)),
            pl.BlockSpec((ROWS_BLK, d_in), lambda i: (i, 0)),
            pl.BlockSpec((2, ROWS_BLK, 16), lambda i: (0, i, 0)),
            pl.BlockSpec((1, d_in), lambda i: (0, 0)),
            pl.BlockSpec((d_in, d_out), lambda i: (0, 0)),
        ],
        out_specs=pl.BlockSpec((ROWS_BLK, d_out), lambda i: (i, 0)),
        out_shape=jax.ShapeDtypeStruct((N, d_out), jnp.float32),
    )(sp, hp, degp, b, w)


def _final_body(sp_ref, hp_ref, deg_ref, b_ref, o_ref):
    dis = _dis_block(deg_ref)
    t = dis * (sp_ref[0] + sp_ref[1] + hp_ref[...])
    t = t[:, :C] + b_ref[...]
    t = t - jnp.max(t, axis=1, keepdims=True)
    e = jnp.exp(t)
    o_ref[...] = e / jnp.sum(e, axis=1, keepdims=True)


def _final(sp, hp, degp, b):
    return pl.pallas_call(
        _final_body,
        grid=(GRID,),
        in_specs=[
            pl.BlockSpec((2, ROWS_BLK, CP), lambda i: (0, i, 0)),
            pl.BlockSpec((ROWS_BLK, CP), lambda i: (i, 0)),
            pl.BlockSpec((2, ROWS_BLK, 16), lambda i: (0, i, 0)),
            pl.BlockSpec((1, C), lambda i: (0, 0)),
        ],
        out_specs=pl.BlockSpec((ROWS_BLK, C), lambda i: (i, 0)),
        out_shape=jax.ShapeDtypeStruct((N, C), jnp.float32),
    )(sp, hp, degp, b)


def kernel(x, edge_index, W1, b1, W2, b2, W3, b3):
    pad = EP - E
    pad_src = jnp.arange(pad, dtype=jnp.int32) % N
    srcp = jnp.concatenate([edge_index[0], pad_src]).reshape(NW, CPT, CH)
    pad_dst = N + (jnp.arange(pad, dtype=jnp.int32) % (NP - N))
    dstp = jnp.concatenate([edge_index[1], pad_dst]).reshape(NW, CPT, CH)
    w3p = jnp.pad(W3, ((0, 0), (0, CP - C)))
    b1r = b1.reshape(1, H1)
    b2r = b2.reshape(1, H2)
    b3r = b3.reshape(1, C)

    degp = _make_hist()(dstp)
    h1p = _in_layer(x, degp, W1)
    s1 = _make_propagate(H1)(h1p, srcp, dstp)
    h2p = _layer(s1, h1p, degp, b1r, W2)
    s2 = _make_propagate(H2)(h2p, srcp, dstp)
    h3p = _layer(s2, h2p, degp, b2r, w3p)
    s3 = _make_propagate(CP)(h3p, srcp, dstp)
    return _final(s3, h3p, degp, b3r)
